# baseline (speedup 1.0000x reference)
import functools

import jax
import jax.numpy as jnp
from jax import lax
from jax.experimental import pallas as pl
from jax.experimental.pallas import tpu as pltpu
from jax.experimental.pallas import tpu_sc as plsc

NC = 2
NS = 16
NW = NC * NS


def _worker_id():
    return lax.axis_index("s") * NC + lax.axis_index("c")


def _remap_kernel(b_per_w, inp_hbm, vmap_hbm, ids_hbm, inp_v, ids_v, sem):
    base = _worker_id() * b_per_w
    pltpu.sync_copy(inp_hbm.at[pl.ds(base, b_per_w)], inp_v)
    pltpu.async_copy(vmap_hbm.at[inp_v], ids_v, sem).wait()
    pltpu.sync_copy(ids_v, ids_hbm.at[pl.ds(base, b_per_w)])


def _iota16():
    return lax.iota(jnp.int32, 16)


def _retile_load(tT_hbm, src_v, col, sem):
    id_base = pl.multiple_of(col * 128, 128)
    pltpu.async_copy(tT_hbm.at[:, pl.ds(id_base, 128)], src_v, sem)


def _retile_transform(src_v, dst_v):
    for lb in range(8):
        lanes = _iota16() + lb * 16
        rows = lax.shift_right_logical(lanes, 1)
        colbase = lax.shift_left(lanes & 1, 6)
        for f8 in range(8):
            vals = [src_v[f8 * 8 + k, lb * 16:(lb + 1) * 16]
                    for k in range(8)]
            for k in range(8):
                plsc.store_scatter(dst_v, [rows, colbase + (f8 * 8 + k)],
                                   vals[k])


def _retile_store(tP_hbm, dst_v, col, sem):
    row_base = pl.multiple_of(col * 64, 64)
    pltpu.async_copy(dst_v, tP_hbm.at[pl.ds(row_base, 64), :], sem)


def _retile_kernel(tT_hbm, tailP_hbm, tP_hbm, src0, src1, dst0, dst1,
                   gsem, ssem):
    w = _worker_id()
    start = w * 244
    srcs = (src0, src1)
    dsts = (dst0, dst1)

    _retile_load(tT_hbm, src0, start, gsem)
    _retile_load(tT_hbm, src1, start + 1, gsem)

    def body(t, carry):
        for b in range(2):
            col = start + 2 * t + b
            pltpu.make_async_copy(
                tT_hbm.at[:, pl.ds(0, 128)], srcs[b], gsem).wait()

            @pl.when(t > 0)
            def _():
                pltpu.make_async_copy(
                    dsts[b], tP_hbm.at[pl.ds(0, 64), :], ssem).wait()

            _retile_transform(srcs[b], dsts[b])
            _retile_store(tP_hbm, dsts[b], col, ssem)

            @pl.when(2 * t + b + 2 < 244)
            def _():
                _retile_load(tT_hbm, srcs[b], col + 2, gsem)
        return carry

    lax.fori_loop(0, 122, body, 0, unroll=False)
    for b in range(2):
        pltpu.make_async_copy(dsts[b], tP_hbm.at[pl.ds(0, 64), :],
                              ssem).wait()

    @pl.when(w >= 28)
    def _():
        col = 32 * 244 + (w - 28)
        pltpu.sync_copy(tT_hbm.at[:, pl.ds(pl.multiple_of(col * 128, 128),
                                           128)], src0)
        _retile_transform(src0, dst0)
        pltpu.sync_copy(dst0, tP_hbm.at[pl.ds(
            pl.multiple_of(col * 64, 64), 64), :])

    @pl.when(w == 0)
    def _():
        pltpu.sync_copy(tailP_hbm, src0.at[0:32, :])
        pltpu.sync_copy(src0.at[0:32, :], tP_hbm.at[pl.ds(499968, 32), :])


def _tc_retile_block(tT_ref, out_ref):
    xt = tT_ref[...].T
    p2, d2 = out_ref.shape
    xt3 = xt.reshape(p2, 2, d2 // 2)
    out_ref[...] = jnp.concatenate([xt3[:, 0, :], xt3[:, 1, :]], axis=1)


def _tc_retile(tT, V, D, CB):
    return pl.pallas_call(
        _tc_retile_block,
        out_shape=jax.ShapeDtypeStruct((V // 2, 2 * D), jnp.float32),
        grid=(pl.cdiv(V, CB),),
        in_specs=[pl.BlockSpec((D, CB), lambda i: (0, i))],
        out_specs=pl.BlockSpec((CB // 2, 2 * D), lambda i: (i, 0)),
    )(tT)


def _gather_prep(ids_v, pid_v, h64_v, off):
    for lb in range(8):
        idv = ids_v[pl.ds(off + lb * 16, 16)]
        pid_v[lb * 16:(lb + 1) * 16] = lax.shift_right_logical(idv, 1)
        h64_v[lb * 16:(lb + 1) * 16] = lax.shift_left(idv & 1, 6)


def _gather_transform(g_v, h64_v, dst_v):
    for lb in range(8):
        rows = _iota16() + lb * 16
        hblk = h64_v[pl.ds(lb * 16, 16)]
        vals = [plsc.load_gather(g_v, [rows, hblk + k])
                for k in range(64)]
        for k in range(64):
            dst_v[k, lb * 16:(lb + 1) * 16] = vals[k]


def _gather_kernel(b_per_w, n_col, ids_hbm, tP_hbm, outT_hbm, ids_v,
                   pid0, pid1, h0, h1, g0, g1, dst0, dst1, gsem, ssem):
    base = _worker_id() * b_per_w
    pltpu.sync_copy(ids_hbm.at[pl.ds(base, b_per_w)], ids_v)
    pids = (pid0, pid1)
    hs = (h0, h1)
    gs = (g0, g1)
    dsts = (dst0, dst1)

    for b in range(2):
        _gather_prep(ids_v, pids[b], hs[b], b * 128)
        pltpu.async_copy(tP_hbm.at[pids[b]], gs[b], gsem)

    def body(t, carry):
        for b in range(2):
            j = 2 * t + b
            off = j * 128
            pltpu.make_async_copy(tP_hbm.at[pids[b]], gs[b], gsem).wait()

            @pl.when(t > 0)
            def _():
                pltpu.make_async_copy(
                    dsts[b], outT_hbm.at[:, pl.ds(0, 128)], ssem).wait()

            _gather_transform(gs[b], hs[b], dsts[b])
            pltpu.async_copy(dsts[b],
                             outT_hbm.at[:, pl.ds(base + off, 128)], ssem)

            @pl.when(j + 2 < n_col)
            def _():
                _gather_prep(ids_v, pids[b], hs[b], off + 256)
                pltpu.async_copy(tP_hbm.at[pids[b]], gs[b], gsem)
        return carry

    lax.fori_loop(0, n_col // 2, body, 0, unroll=False)
    for b in range(2):
        pltpu.make_async_copy(dsts[b], outT_hbm.at[:, pl.ds(0, 128)],
                              ssem).wait()


def kernel(input, vocab_map, table):
    B = input.shape[0]
    V, D = table.shape
    b_per_w = B // NW
    n_col = b_per_w // 128
    mesh = plsc.VectorSubcoreMesh(core_axis_name="c", subcore_axis_name="s")

    ids = pl.kernel(
        functools.partial(_remap_kernel, b_per_w),
        out_type=jax.ShapeDtypeStruct((B,), jnp.int32),
        mesh=mesh,
        compiler_params=pltpu.CompilerParams(use_tc_tiling_on_sc=False),
        scratch_types=[
            pltpu.VMEM((b_per_w,), jnp.int32),
            pltpu.VMEM((b_per_w,), jnp.int32),
            pltpu.SemaphoreType.DMA,
        ],
    )(input, vocab_map)

    tableP = _tc_retile(table.T, V, D, 8192)

    outT = pl.kernel(
        functools.partial(_gather_kernel, b_per_w, n_col),
        out_type=jax.ShapeDtypeStruct((D, B), jnp.float32),
        mesh=mesh,
        compiler_params=pltpu.CompilerParams(use_tc_tiling_on_sc=True,
                                            needs_layout_passes=False),
        scratch_types=[
            pltpu.VMEM((b_per_w,), jnp.int32),
            pltpu.VMEM((128,), jnp.int32),
            pltpu.VMEM((128,), jnp.int32),
            pltpu.VMEM((128,), jnp.int32),
            pltpu.VMEM((128,), jnp.int32),
            pltpu.VMEM((128, 128), jnp.float32),
            pltpu.VMEM((128, 128), jnp.float32),
            pltpu.VMEM((D, 128), jnp.float32),
            pltpu.VMEM((D, 128), jnp.float32),
            pltpu.SemaphoreType.DMA,
            pltpu.SemaphoreType.DMA,
        ],
    )(ids, tableP)

    return outT.T


# speedup vs baseline: 1.1108x; 1.1108x over previous
import functools

import jax
import jax.numpy as jnp
from jax import lax
from jax.experimental import pallas as pl
from jax.experimental.pallas import tpu as pltpu
from jax.experimental.pallas import tpu_sc as plsc

NC = 2
NS = 16
NW = NC * NS


def _worker_id():
    return lax.axis_index("s") * NC + lax.axis_index("c")


def _remap_kernel(b_per_w, inp_hbm, vmap_hbm, ids_hbm, inp_v, ids_v, sem):
    base = _worker_id() * b_per_w
    pltpu.sync_copy(inp_hbm.at[pl.ds(base, b_per_w)], inp_v)
    pltpu.async_copy(vmap_hbm.at[inp_v], ids_v, sem).wait()
    pltpu.sync_copy(ids_v, ids_hbm.at[pl.ds(base, b_per_w)])


def _iota16():
    return lax.iota(jnp.int32, 16)


def _retile_load(tT_hbm, src_v, col, sem):
    id_base = pl.multiple_of(col * 128, 128)
    pltpu.async_copy(tT_hbm.at[:, pl.ds(id_base, 128)], src_v, sem)


def _retile_transform(src_v, dst_v):
    for lb in range(8):
        lanes = _iota16() + lb * 16
        rows = lax.shift_right_logical(lanes, 1)
        colbase = lax.shift_left(lanes & 1, 6)
        for f8 in range(8):
            vals = [src_v[f8 * 8 + k, lb * 16:(lb + 1) * 16]
                    for k in range(8)]
            for k in range(8):
                plsc.store_scatter(dst_v, [rows, colbase + (f8 * 8 + k)],
                                   vals[k])


def _retile_store(tP_hbm, dst_v, col, sem):
    row_base = pl.multiple_of(col * 64, 64)
    pltpu.async_copy(dst_v, tP_hbm.at[pl.ds(row_base, 64), :], sem)


def _retile_kernel(tT_hbm, tailP_hbm, tP_hbm, src0, src1, dst0, dst1,
                   gsem, ssem):
    w = _worker_id()
    start = w * 244
    srcs = (src0, src1)
    dsts = (dst0, dst1)

    _retile_load(tT_hbm, src0, start, gsem)
    _retile_load(tT_hbm, src1, start + 1, gsem)

    def body(t, carry):
        for b in range(2):
            col = start + 2 * t + b
            pltpu.make_async_copy(
                tT_hbm.at[:, pl.ds(0, 128)], srcs[b], gsem).wait()

            @pl.when(t > 0)
            def _():
                pltpu.make_async_copy(
                    dsts[b], tP_hbm.at[pl.ds(0, 64), :], ssem).wait()

            _retile_transform(srcs[b], dsts[b])
            _retile_store(tP_hbm, dsts[b], col, ssem)

            @pl.when(2 * t + b + 2 < 244)
            def _():
                _retile_load(tT_hbm, srcs[b], col + 2, gsem)
        return carry

    lax.fori_loop(0, 122, body, 0, unroll=False)
    for b in range(2):
        pltpu.make_async_copy(dsts[b], tP_hbm.at[pl.ds(0, 64), :],
                              ssem).wait()

    @pl.when(w >= 28)
    def _():
        col = 32 * 244 + (w - 28)
        pltpu.sync_copy(tT_hbm.at[:, pl.ds(pl.multiple_of(col * 128, 128),
                                           128)], src0)
        _retile_transform(src0, dst0)
        pltpu.sync_copy(dst0, tP_hbm.at[pl.ds(
            pl.multiple_of(col * 64, 64), 64), :])

    @pl.when(w == 0)
    def _():
        pltpu.sync_copy(tailP_hbm, src0.at[0:32, :])
        pltpu.sync_copy(src0.at[0:32, :], tP_hbm.at[pl.ds(499968, 32), :])


def _tc_retile_block(tT_ref, out_ref):
    xt = tT_ref[...].T
    p2, d2 = out_ref.shape
    xt3 = xt.reshape(p2, 2, d2 // 2)
    out_ref[...] = jnp.concatenate([xt3[:, 0, :], xt3[:, 1, :]], axis=1)


def _tc_retile(tT, V, D, CB):
    return pl.pallas_call(
        _tc_retile_block,
        out_shape=jax.ShapeDtypeStruct((V // 2, 2 * D), jnp.float32),
        grid=(pl.cdiv(V, CB),),
        in_specs=[pl.BlockSpec((D, CB), lambda i: (0, i))],
        out_specs=pl.BlockSpec((CB // 2, 2 * D), lambda i: (i, 0)),
    )(tT)


def _gather_prep(ids_v, pid_v, h64_v, off):
    for lb in range(8):
        idv = ids_v[pl.ds(off + lb * 16, 16)]
        pid_v[lb * 16:(lb + 1) * 16] = lax.shift_right_logical(idv, 1)
        h64_v[lb * 16:(lb + 1) * 16] = lax.shift_left(idv & 1, 6)


def _gather_transform(g_v, h64_v, dst_v):
    for lb in range(8):
        rows = _iota16() + lb * 16
        hblk = h64_v[pl.ds(lb * 16, 16)]
        for f32b in range(2):
            vals = [plsc.load_gather(g_v, [rows, hblk + (f32b * 32 + k)])
                    for k in range(32)]
            for k in range(32):
                f = f32b * 32 + k
                dst_v[f, lb * 16:(lb + 1) * 16] = vals[k]


def _gather_kernel(b_per_w, n_col, ids_hbm, tP_hbm, outT_hbm, ids_v,
                   pid0, pid1, h0, h1, g0, g1, dst0, dst1, gsem, ssem):
    base = _worker_id() * b_per_w
    pltpu.sync_copy(ids_hbm.at[pl.ds(base, b_per_w)], ids_v)
    pids = (pid0, pid1)
    hs = (h0, h1)
    gs = (g0, g1)
    dsts = (dst0, dst1)

    for b in range(2):
        _gather_prep(ids_v, pids[b], hs[b], b * 128)
        pltpu.async_copy(tP_hbm.at[pids[b]], gs[b], gsem)

    def body(t, carry):
        for b in range(2):
            j = 2 * t + b
            off = j * 128
            pltpu.make_async_copy(tP_hbm.at[pids[b]], gs[b], gsem).wait()

            @pl.when(t > 0)
            def _():
                pltpu.make_async_copy(
                    dsts[b], outT_hbm.at[:, pl.ds(0, 128)], ssem).wait()

            _gather_transform(gs[b], hs[b], dsts[b])
            pltpu.async_copy(dsts[b],
                             outT_hbm.at[:, pl.ds(base + off, 128)], ssem)

            @pl.when(j + 2 < n_col)
            def _():
                _gather_prep(ids_v, pids[b], hs[b], off + 256)
                pltpu.async_copy(tP_hbm.at[pids[b]], gs[b], gsem)
        return carry

    lax.fori_loop(0, n_col // 2, body, 0, unroll=False)
    for b in range(2):
        pltpu.make_async_copy(dsts[b], outT_hbm.at[:, pl.ds(0, 128)],
                              ssem).wait()


def kernel(input, vocab_map, table):
    B = input.shape[0]
    V, D = table.shape
    b_per_w = B // NW
    n_col = b_per_w // 128
    mesh = plsc.VectorSubcoreMesh(core_axis_name="c", subcore_axis_name="s")

    ids = pl.kernel(
        functools.partial(_remap_kernel, b_per_w),
        out_type=jax.ShapeDtypeStruct((B,), jnp.int32),
        mesh=mesh,
        compiler_params=pltpu.CompilerParams(use_tc_tiling_on_sc=False),
        scratch_types=[
            pltpu.VMEM((b_per_w,), jnp.int32),
            pltpu.VMEM((b_per_w,), jnp.int32),
            pltpu.SemaphoreType.DMA,
        ],
    )(input, vocab_map)

    tableP = _tc_retile(table.T, V, D, 8192)

    outT = pl.kernel(
        functools.partial(_gather_kernel, b_per_w, n_col),
        out_type=jax.ShapeDtypeStruct((D, B), jnp.float32),
        mesh=mesh,
        compiler_params=pltpu.CompilerParams(use_tc_tiling_on_sc=True,
                                            needs_layout_passes=False),
        scratch_types=[
            pltpu.VMEM((b_per_w,), jnp.int32),
            pltpu.VMEM((128,), jnp.int32),
            pltpu.VMEM((128,), jnp.int32),
            pltpu.VMEM((128,), jnp.int32),
            pltpu.VMEM((128,), jnp.int32),
            pltpu.VMEM((128, 128), jnp.float32),
            pltpu.VMEM((128, 128), jnp.float32),
            pltpu.VMEM((D, 128), jnp.float32),
            pltpu.VMEM((D, 128), jnp.float32),
            pltpu.SemaphoreType.DMA,
            pltpu.SemaphoreType.DMA,
        ],
    )(ids, tableP)

    return outT.T


# speedup vs baseline: 1.1171x; 1.0056x over previous
import functools

import jax
import jax.numpy as jnp
from jax import lax
from jax.experimental import pallas as pl
from jax.experimental.pallas import tpu as pltpu
from jax.experimental.pallas import tpu_sc as plsc

NC = 2
NS = 16
NW = NC * NS


def _worker_id():
    return lax.axis_index("s") * NC + lax.axis_index("c")


def _remap_kernel(b_per_w, inp_hbm, vmap_hbm, ids_hbm, inp_v, ids_v, sem):
    base = _worker_id() * b_per_w
    pltpu.sync_copy(inp_hbm.at[pl.ds(base, b_per_w)], inp_v)
    pltpu.async_copy(vmap_hbm.at[inp_v], ids_v, sem).wait()
    pltpu.sync_copy(ids_v, ids_hbm.at[pl.ds(base, b_per_w)])


def _iota16():
    return lax.iota(jnp.int32, 16)


def _retile_load(tT_hbm, src_v, col, sem):
    id_base = pl.multiple_of(col * 128, 128)
    pltpu.async_copy(tT_hbm.at[:, pl.ds(id_base, 128)], src_v, sem)


def _retile_transform(src_v, dst_v):
    for lb in range(8):
        lanes = _iota16() + lb * 16
        rows = lax.shift_right_logical(lanes, 1)
        colbase = lax.shift_left(lanes & 1, 6)
        for f8 in range(8):
            vals = [src_v[f8 * 8 + k, lb * 16:(lb + 1) * 16]
                    for k in range(8)]
            for k in range(8):
                plsc.store_scatter(dst_v, [rows, colbase + (f8 * 8 + k)],
                                   vals[k])


def _retile_store(tP_hbm, dst_v, col, sem):
    row_base = pl.multiple_of(col * 64, 64)
    pltpu.async_copy(dst_v, tP_hbm.at[pl.ds(row_base, 64), :], sem)


def _retile_kernel(tT_hbm, tailP_hbm, tP_hbm, src0, src1, dst0, dst1,
                   gsem, ssem):
    w = _worker_id()
    start = w * 244
    srcs = (src0, src1)
    dsts = (dst0, dst1)

    _retile_load(tT_hbm, src0, start, gsem)
    _retile_load(tT_hbm, src1, start + 1, gsem)

    def body(t, carry):
        for b in range(2):
            col = start + 2 * t + b
            pltpu.make_async_copy(
                tT_hbm.at[:, pl.ds(0, 128)], srcs[b], gsem).wait()

            @pl.when(t > 0)
            def _():
                pltpu.make_async_copy(
                    dsts[b], tP_hbm.at[pl.ds(0, 64), :], ssem).wait()

            _retile_transform(srcs[b], dsts[b])
            _retile_store(tP_hbm, dsts[b], col, ssem)

            @pl.when(2 * t + b + 2 < 244)
            def _():
                _retile_load(tT_hbm, srcs[b], col + 2, gsem)
        return carry

    lax.fori_loop(0, 122, body, 0, unroll=False)
    for b in range(2):
        pltpu.make_async_copy(dsts[b], tP_hbm.at[pl.ds(0, 64), :],
                              ssem).wait()

    @pl.when(w >= 28)
    def _():
        col = 32 * 244 + (w - 28)
        pltpu.sync_copy(tT_hbm.at[:, pl.ds(pl.multiple_of(col * 128, 128),
                                           128)], src0)
        _retile_transform(src0, dst0)
        pltpu.sync_copy(dst0, tP_hbm.at[pl.ds(
            pl.multiple_of(col * 64, 64), 64), :])

    @pl.when(w == 0)
    def _():
        pltpu.sync_copy(tailP_hbm, src0.at[0:32, :])
        pltpu.sync_copy(src0.at[0:32, :], tP_hbm.at[pl.ds(499968, 32), :])


def _tc_retile_block(tT_ref, out_ref):
    xt = tT_ref[...].T
    p2, d2 = out_ref.shape
    xt3 = xt.reshape(p2, 2, d2 // 2)
    out_ref[...] = jnp.concatenate([xt3[:, 0, :], xt3[:, 1, :]], axis=1)


def _tc_retile(tT, V, D, CB):
    return pl.pallas_call(
        _tc_retile_block,
        out_shape=jax.ShapeDtypeStruct((V // 2, 2 * D), jnp.float32),
        grid=(pl.cdiv(V, CB),),
        in_specs=[pl.BlockSpec((D, CB), lambda i: (0, i))],
        out_specs=pl.BlockSpec((CB // 2, 2 * D), lambda i: (i, 0)),
    )(tT)


def _gather_prep(ids_v, pid_v, h64_v, off):
    for lb in range(8):
        idv = ids_v[pl.ds(off + lb * 16, 16)]
        pid_v[lb * 16:(lb + 1) * 16] = lax.shift_right_logical(idv, 1)
        h64_v[lb * 16:(lb + 1) * 16] = lax.shift_left(idv & 1, 6)


def _gather_transform(g_v, h64_v, dst_v):
    for lb in range(8):
        rows = _iota16() + lb * 16
        hblk = h64_v[pl.ds(lb * 16, 16)]
        for f32b in range(2):
            vals = [plsc.load_gather(g_v, [rows, hblk + (f32b * 32 + k)])
                    for k in range(32)]
            for k in range(32):
                f = f32b * 32 + k
                dst_v[f, lb * 16:(lb + 1) * 16] = vals[k]


def _gather_kernel(b_per_w, n_col, ids_hbm, tP_hbm, outT_hbm, ids_v,
                   pid0, pid1, h0, h1, g0, g1, dst0, dst1, gsem, ssem):
    base = _worker_id() * b_per_w
    pltpu.sync_copy(ids_hbm.at[pl.ds(base, b_per_w)], ids_v)
    pids = (pid0, pid1)
    hs = (h0, h1)
    gs = (g0, g1)
    dsts = (dst0, dst1)

    for b in range(2):
        _gather_prep(ids_v, pids[b], hs[b], b * 128)
        pltpu.async_copy(tP_hbm.at[pids[b]], gs[b], gsem)

    def body(t, carry):
        for b in range(2):
            j = 2 * t + b
            off = j * 128
            pltpu.make_async_copy(tP_hbm.at[pids[b]], gs[b], gsem).wait()

            @pl.when(t > 0)
            def _():
                pltpu.make_async_copy(
                    dsts[b], outT_hbm.at[:, pl.ds(0, 128)], ssem).wait()

            _gather_transform(gs[b], hs[b], dsts[b])
            pltpu.async_copy(dsts[b],
                             outT_hbm.at[:, pl.ds(base + off, 128)], ssem)

            @pl.when(j + 2 < n_col)
            def _():
                _gather_prep(ids_v, pids[b], hs[b], off + 256)
                pltpu.async_copy(tP_hbm.at[pids[b]], gs[b], gsem)
        return carry

    lax.fori_loop(0, n_col // 2, body, 0, unroll=False)
    for b in range(2):
        pltpu.make_async_copy(dsts[b], outT_hbm.at[:, pl.ds(0, 128)],
                              ssem).wait()


def kernel(input, vocab_map, table):
    B = input.shape[0]
    V, D = table.shape
    b_per_w = B // NW
    n_col = b_per_w // 128
    mesh = plsc.VectorSubcoreMesh(core_axis_name="c", subcore_axis_name="s")

    ids = pl.kernel(
        functools.partial(_remap_kernel, b_per_w),
        out_type=jax.ShapeDtypeStruct((B,), jnp.int32),
        mesh=mesh,
        compiler_params=pltpu.CompilerParams(use_tc_tiling_on_sc=False),
        scratch_types=[
            pltpu.VMEM((b_per_w,), jnp.int32),
            pltpu.VMEM((b_per_w,), jnp.int32),
            pltpu.SemaphoreType.DMA,
        ],
    )(input, vocab_map)

    tableP = _tc_retile(table.T, V, D, 16384)

    outT = pl.kernel(
        functools.partial(_gather_kernel, b_per_w, n_col),
        out_type=jax.ShapeDtypeStruct((D, B), jnp.float32),
        mesh=mesh,
        compiler_params=pltpu.CompilerParams(use_tc_tiling_on_sc=True,
                                            needs_layout_passes=False),
        scratch_types=[
            pltpu.VMEM((b_per_w,), jnp.int32),
            pltpu.VMEM((128,), jnp.int32),
            pltpu.VMEM((128,), jnp.int32),
            pltpu.VMEM((128,), jnp.int32),
            pltpu.VMEM((128,), jnp.int32),
            pltpu.VMEM((128, 128), jnp.float32),
            pltpu.VMEM((128, 128), jnp.float32),
            pltpu.VMEM((D, 128), jnp.float32),
            pltpu.VMEM((D, 128), jnp.float32),
            pltpu.SemaphoreType.DMA,
            pltpu.SemaphoreType.DMA,
        ],
    )(ids, tableP)

    return outT.T
